# Initial kernel scaffold; baseline (speedup 1.0000x reference)
import jax
import jax.numpy as jnp
from jax.experimental import pallas as pl


def kernel(x, edge_index, batch, pos, a, b, params, epoch):
    raise NotImplementedError("write your pallas kernel here")


# speedup vs baseline: 3.6693x; 3.6693x over previous
import functools

import jax
import jax.numpy as jnp
from jax import lax
from jax.experimental import pallas as pl
from jax.experimental.pallas import tpu as pltpu
from jax.experimental.pallas import tpu_sc as plsc

N = 10000
E = 320000
D = 128
HID = 128
G = 64
L = 3
EPS = 1e-7

NC = 2
NS = 16
LANES = 16
NW = NC * NS
CPT = D // NW
EPT = E // NW

CH_SS = 16000
CH_E = 5000

_sc_mesh = plsc.VectorSubcoreMesh(core_axis_name="c", subcore_axis_name="s",
                                  num_cores=NC, num_subcores=NS)


def _enc_body(x_ref, w_ref, b_ref, out_ref):
    h = lax.dot_general(w_ref[...], x_ref[...], (((0,), (1,)), ((), ())),
                        preferred_element_type=jnp.float32)
    out_ref[...] = jnp.maximum(h + b_ref[...], 0.0)


def _encode(x, w, b):
    return pl.pallas_call(
        _enc_body,
        out_shape=jax.ShapeDtypeStruct((D, N), jnp.float32),
    )(x, w, b.reshape(D, 1))


def _gin_body(h_ref, m_ref, w_ref, b_ref, eps_ref, out_ref):
    t = (1.0 + eps_ref[0, 0]) * h_ref[...] + m_ref[...]
    o = lax.dot_general(w_ref[...], t, (((0,), (0,)), ((), ())),
                        preferred_element_type=jnp.float32)
    out_ref[...] = jnp.maximum(o + b_ref[...], 0.0)


def _gin(hT, msgT, w, b, eps):
    return pl.pallas_call(
        _gin_body,
        out_shape=jax.ShapeDtypeStruct((D, N), jnp.float32),
    )(hT, msgT, w, b.reshape(D, 1), eps.reshape(1, 1))


def _head_body(h_ref, batch_ref, w1_ref, b1_ref, w2_ref, b2_ref,
               gf_ref, zi_ref):
    hb = h_ref[...]
    bt = batch_ref[...]
    iot = lax.broadcasted_iota(jnp.int32, (N, G), 1)
    oh = (iot == bt).astype(jnp.float32)

    gsum = lax.dot_general(oh, hb, (((0,), (1,)), ((), ())),
                           preferred_element_type=jnp.float32)
    cnt = lax.dot_general(oh, jnp.ones((N, 1), jnp.float32),
                          (((0,), (0,)), ((), ())),
                          preferred_element_type=jnp.float32)
    gf_ref[...] = gsum / jnp.maximum(cnt, 1.0)

    z1 = jnp.maximum(
        lax.dot_general(w1_ref[...], hb, (((0,), (0,)), ((), ())),
                        preferred_element_type=jnp.float32) + b1_ref[...], 0.0)
    zi_ref[...] = lax.dot_general(w2_ref[...], z1, (((0,), (0,)), ((), ())),
                                  preferred_element_type=jnp.float32) + b2_ref[...]


def _head(hT, batch_col, w1, b1, w2p, b2p):
    return pl.pallas_call(
        _head_body,
        out_shape=[
            jax.ShapeDtypeStruct((G, D), jnp.float32),
            jax.ShapeDtypeStruct((8, N), jnp.float32),
        ],
    )(hT, batch_col, w1, b1.reshape(HID, 1), w2p, b2p.reshape(8, 1))


def _loss_body(d2z_ref, d2q_ref, pos_ref, a_ref, b_ref, ce_ref, pl_ref):
    am = a_ref[0, 0]
    bm = b_ref[0, 0]

    def prob(d2):
        t = d2 + 1e-12
        p = 1.0 / (1.0 + am * jnp.exp(bm * jnp.log(t)))
        return jnp.clip(p, EPS, 1.0 - EPS)

    p = prob(d2z_ref[...])
    q = prob(d2q_ref[...])
    ce = -(q * jnp.log(p) + (1.0 - q) * jnp.log(1.0 - p))
    ce_ref[...] = (jnp.sum(ce) / E).reshape(1, 1)
    pl_ref[...] = (jnp.sum(pos_ref[...] ** 2) / (N * 3)).reshape(1, 1)


def _loss(d2z, d2q, pos, a, b):
    return pl.pallas_call(
        _loss_body,
        out_shape=[
            jax.ShapeDtypeStruct((1, 1), jnp.float32),
            jax.ShapeDtypeStruct((1, 1), jnp.float32),
        ],
    )(d2z.reshape(E // 128, 128), d2q.reshape(E // 128, 128), pos,
      a.reshape(1, 1), b.reshape(1, 1))


def _segsum_body(hT_hbm, src_hbm, dst_hbm, out_hbm,
                 h0, h1, h2, h3, m0, m1, m2, m3, sbuf, dbuf):
    wid = lax.axis_index("s") * NC + lax.axis_index("c")
    base = wid * CPT
    hcols = (h0, h1, h2, h3)
    mcols = (m0, m1, m2, m3)
    for c in range(CPT):
        pltpu.sync_copy(hT_hbm.at[base + c], hcols[c])

    def zero_body(i, _):
        z = jnp.zeros((LANES,), jnp.float32)
        for c in range(CPT):
            mcols[c][pl.ds(i * LANES, LANES)] = z
        return 0

    lax.fori_loop(0, N // LANES, zero_body, 0)

    def chunk_body(k, _):
        pltpu.sync_copy(src_hbm.at[pl.ds(k * CH_SS, CH_SS)], sbuf)
        pltpu.sync_copy(dst_hbm.at[pl.ds(k * CH_SS, CH_SS)], dbuf)

        def grp_body(j, _):
            s = sbuf[pl.ds(j * LANES, LANES)]
            d = dbuf[pl.ds(j * LANES, LANES)]
            for c in range(CPT):
                v = plsc.load_gather(hcols[c], [s])
                plsc.addupdate_scatter(mcols[c], [d], v)
            return 0

        lax.fori_loop(0, CH_SS // LANES, grp_body, 0)
        return 0

    lax.fori_loop(0, E // CH_SS, chunk_body, 0)
    for c in range(CPT):
        pltpu.sync_copy(mcols[c], out_hbm.at[base + c])


_segsum = functools.partial(
    pl.kernel,
    _segsum_body,
    out_type=jax.ShapeDtypeStruct((D, N), jnp.float32),
    mesh=_sc_mesh,
    compiler_params=pltpu.CompilerParams(needs_layout_passes=False,
                                        use_tc_tiling_on_sc=False),
    scratch_types=(
        [pltpu.VMEM((N,), jnp.float32)] * 8
        + [pltpu.VMEM((CH_SS,), jnp.int32)] * 2
    ),
)()


def _edge_d2_body(zi_hbm, pos_hbm, src_hbm, dst_hbm, d2z_hbm, d2q_hbm,
                  z0, z1, z2, p0, p1, p2, sbuf, dbuf, oz, oq):
    wid = lax.axis_index("s") * NC + lax.axis_index("c")
    zcols = (z0, z1, z2)
    pcols = (p0, p1, p2)
    for c in range(3):
        pltpu.sync_copy(zi_hbm.at[c], zcols[c])
        pltpu.sync_copy(pos_hbm.at[c], pcols[c])
    base = wid * EPT

    def chunk_body(k, _):
        off = base + k * CH_E
        pltpu.sync_copy(src_hbm.at[pl.ds(off, CH_E)], sbuf)
        pltpu.sync_copy(dst_hbm.at[pl.ds(off, CH_E)], dbuf)

        def grp_body(j, _):
            s = sbuf[pl.ds(j * LANES, LANES)]
            d = dbuf[pl.ds(j * LANES, LANES)]
            az = jnp.zeros((LANES,), jnp.float32)
            aq = jnp.zeros((LANES,), jnp.float32)
            for c in range(3):
                dz = plsc.load_gather(zcols[c], [s]) - plsc.load_gather(zcols[c], [d])
                az = az + dz * dz
                dq = plsc.load_gather(pcols[c], [s]) - plsc.load_gather(pcols[c], [d])
                aq = aq + dq * dq
            oz[pl.ds(j * LANES, LANES)] = az
            oq[pl.ds(j * LANES, LANES)] = aq
            return 0

        lax.fori_loop(0, CH_E // LANES, grp_body, 0)
        pltpu.sync_copy(oz, d2z_hbm.at[pl.ds(off, CH_E)])
        pltpu.sync_copy(oq, d2q_hbm.at[pl.ds(off, CH_E)])
        return 0

    lax.fori_loop(0, EPT // CH_E, chunk_body, 0)


_edge_d2 = functools.partial(
    pl.kernel,
    _edge_d2_body,
    out_type=(
        jax.ShapeDtypeStruct((E,), jnp.float32),
        jax.ShapeDtypeStruct((E,), jnp.float32),
    ),
    mesh=_sc_mesh,
    compiler_params=pltpu.CompilerParams(needs_layout_passes=False,
                                        use_tc_tiling_on_sc=False),
    scratch_types=(
        [pltpu.VMEM((N,), jnp.float32)] * 6
        + [pltpu.VMEM((CH_E,), jnp.int32)] * 2
        + [pltpu.VMEM((CH_E,), jnp.float32)] * 2
    ),
)()


def kernel(x, edge_index, batch, pos, a, b, params, epoch):
    src = edge_index[0]
    dst = edge_index[1]
    p = params

    hT = _encode(x, p['enc_W'], p['enc_b'])
    for l in range(L):
        msgT = _segsum(hT, src, dst)
        hT = _gin(hT, msgT, p['gin_W'][l], p['gin_b'][l], p['gin_eps'][l])

    w2p = jnp.zeros((HID, 8), jnp.float32).at[:, :3].set(p['z_W2'])
    b2p = jnp.zeros((8,), jnp.float32).at[:3].set(p['z_b2'])
    graph_feat, ziT8 = _head(hT, batch.reshape(N, 1), p['z_W1'], p['z_b1'],
                             w2p, b2p)

    d2z, d2q = _edge_d2(ziT8[:3], pos.T, src, dst)
    ce, ploss = _loss(d2z, d2q, pos, a, b)

    pred_pos = jnp.zeros((N, 3), x.dtype)
    return pred_pos, graph_feat, ploss.reshape(()), ce.reshape(())


# speedup vs baseline: 3.6877x; 1.0050x over previous
import functools

import jax
import jax.numpy as jnp
from jax import lax
from jax.experimental import pallas as pl
from jax.experimental.pallas import tpu as pltpu
from jax.experimental.pallas import tpu_sc as plsc

N = 10000
E = 320000
D = 128
HID = 128
G = 64
L = 3
EPS = 1e-7

NC = 2
NS = 16
LANES = 16
NW = NC * NS
CPT = D // NW
EPT = E // NW

CH_SS = 16000
CH_E = 5000

_sc_mesh = plsc.VectorSubcoreMesh(core_axis_name="c", subcore_axis_name="s",
                                  num_cores=NC, num_subcores=NS)


def _enc_body(x_ref, w_ref, b_ref, out_ref):
    h = lax.dot_general(w_ref[...], x_ref[...], (((0,), (1,)), ((), ())),
                        preferred_element_type=jnp.float32)
    out_ref[...] = jnp.maximum(h + b_ref[...], 0.0)


def _encode(x, w, b):
    return pl.pallas_call(
        _enc_body,
        out_shape=jax.ShapeDtypeStruct((D, N), jnp.float32),
    )(x, w, b.reshape(D, 1))


def _gin_body(h_ref, m_ref, w_ref, b_ref, eps_ref, out_ref):
    t = (1.0 + eps_ref[0, 0]) * h_ref[...] + m_ref[...]
    o = lax.dot_general(w_ref[...], t, (((0,), (0,)), ((), ())),
                        preferred_element_type=jnp.float32)
    out_ref[...] = jnp.maximum(o + b_ref[...], 0.0)


def _gin(hT, msgT, w, b, eps):
    return pl.pallas_call(
        _gin_body,
        out_shape=jax.ShapeDtypeStruct((D, N), jnp.float32),
    )(hT, msgT, w, b.reshape(D, 1), eps.reshape(1, 1))


def _head_body(h_ref, batch_ref, w1_ref, b1_ref, w2_ref, b2_ref,
               gf_ref, zi_ref):
    hb = h_ref[...]
    bt = batch_ref[...]
    iot = lax.broadcasted_iota(jnp.int32, (N, G), 1)
    oh = (iot == bt).astype(jnp.float32)

    gsum = lax.dot_general(oh, hb, (((0,), (1,)), ((), ())),
                           preferred_element_type=jnp.float32)
    cnt = lax.dot_general(oh, jnp.ones((N, 1), jnp.float32),
                          (((0,), (0,)), ((), ())),
                          preferred_element_type=jnp.float32)
    gf_ref[...] = gsum / jnp.maximum(cnt, 1.0)

    z1 = jnp.maximum(
        lax.dot_general(w1_ref[...], hb, (((0,), (0,)), ((), ())),
                        preferred_element_type=jnp.float32) + b1_ref[...], 0.0)
    zi_ref[...] = lax.dot_general(w2_ref[...], z1, (((0,), (0,)), ((), ())),
                                  preferred_element_type=jnp.float32) + b2_ref[...]


def _head(hT, batch_col, w1, b1, w2p, b2p):
    return pl.pallas_call(
        _head_body,
        out_shape=[
            jax.ShapeDtypeStruct((G, D), jnp.float32),
            jax.ShapeDtypeStruct((8, N), jnp.float32),
        ],
    )(hT, batch_col, w1, b1.reshape(HID, 1), w2p, b2p.reshape(8, 1))


def _loss_body(d2z_ref, d2q_ref, pos_ref, a_ref, b_ref, ce_ref, pl_ref):
    am = a_ref[0, 0]
    bm = b_ref[0, 0]

    def prob(d2):
        t = d2 + 1e-12
        p = 1.0 / (1.0 + am * jnp.exp(bm * jnp.log(t)))
        return jnp.clip(p, EPS, 1.0 - EPS)

    p = prob(d2z_ref[...])
    q = prob(d2q_ref[...])
    ce = -(q * jnp.log(p) + (1.0 - q) * jnp.log(1.0 - p))
    ce_ref[...] = (jnp.sum(ce) / E).reshape(1, 1)
    pl_ref[...] = (jnp.sum(pos_ref[...] ** 2) / (N * 3)).reshape(1, 1)


def _loss(d2z, d2q, pos, a, b):
    return pl.pallas_call(
        _loss_body,
        out_shape=[
            jax.ShapeDtypeStruct((1, 1), jnp.float32),
            jax.ShapeDtypeStruct((1, 1), jnp.float32),
        ],
    )(d2z.reshape(E // 128, 128), d2q.reshape(E // 128, 128), pos,
      a.reshape(1, 1), b.reshape(1, 1))


def _segsum_body(hT_hbm, src_hbm, dst_hbm, out_hbm,
                 h0, h1, h2, h3, m0, m1, m2, m3, sbuf, dbuf):
    wid = lax.axis_index("s") * NC + lax.axis_index("c")
    base = wid * CPT
    hcols = (h0, h1, h2, h3)
    mcols = (m0, m1, m2, m3)
    for c in range(CPT):
        pltpu.sync_copy(hT_hbm.at[base + c], hcols[c])

    def zero_body(i, _):
        z = jnp.zeros((LANES,), jnp.float32)
        for c in range(CPT):
            mcols[c][pl.ds(i * LANES, LANES)] = z
        return 0

    lax.fori_loop(0, N // LANES, zero_body, 0)

    def chunk_body(k, _):
        pltpu.sync_copy(src_hbm.at[pl.ds(k * CH_SS, CH_SS)], sbuf)
        pltpu.sync_copy(dst_hbm.at[pl.ds(k * CH_SS, CH_SS)], dbuf)

        def grp_body(j, _):
            for u in range(4):
                s = sbuf[pl.ds(j * (4 * LANES) + u * LANES, LANES)]
                d = dbuf[pl.ds(j * (4 * LANES) + u * LANES, LANES)]
                for c in range(CPT):
                    v = plsc.load_gather(hcols[c], [s])
                    plsc.addupdate_scatter(mcols[c], [d], v)
            return 0

        lax.fori_loop(0, CH_SS // (4 * LANES), grp_body, 0)
        return 0

    lax.fori_loop(0, E // CH_SS, chunk_body, 0)
    for c in range(CPT):
        pltpu.sync_copy(mcols[c], out_hbm.at[base + c])


_segsum = functools.partial(
    pl.kernel,
    _segsum_body,
    out_type=jax.ShapeDtypeStruct((D, N), jnp.float32),
    mesh=_sc_mesh,
    compiler_params=pltpu.CompilerParams(needs_layout_passes=False,
                                        use_tc_tiling_on_sc=False),
    scratch_types=(
        [pltpu.VMEM((N,), jnp.float32)] * 8
        + [pltpu.VMEM((CH_SS,), jnp.int32)] * 2
    ),
)()


def _edge_d2_body(zi_hbm, pos_hbm, src_hbm, dst_hbm, d2z_hbm, d2q_hbm,
                  z0, z1, z2, p0, p1, p2, sbuf, dbuf, oz, oq):
    wid = lax.axis_index("s") * NC + lax.axis_index("c")
    zcols = (z0, z1, z2)
    pcols = (p0, p1, p2)
    for c in range(3):
        pltpu.sync_copy(zi_hbm.at[c], zcols[c])
        pltpu.sync_copy(pos_hbm.at[c], pcols[c])
    base = wid * EPT

    def chunk_body(k, _):
        off = base + k * CH_E
        pltpu.sync_copy(src_hbm.at[pl.ds(off, CH_E)], sbuf)
        pltpu.sync_copy(dst_hbm.at[pl.ds(off, CH_E)], dbuf)

        def grp_body(j, _):
            for u in range(2):
                jj = j * (2 * LANES) + u * LANES
                s = sbuf[pl.ds(jj, LANES)]
                d = dbuf[pl.ds(jj, LANES)]
                az = jnp.zeros((LANES,), jnp.float32)
                aq = jnp.zeros((LANES,), jnp.float32)
                for c in range(3):
                    dz = plsc.load_gather(zcols[c], [s]) - plsc.load_gather(zcols[c], [d])
                    az = az + dz * dz
                    dq = plsc.load_gather(pcols[c], [s]) - plsc.load_gather(pcols[c], [d])
                    aq = aq + dq * dq
                oz[pl.ds(jj, LANES)] = az
                oq[pl.ds(jj, LANES)] = aq
            return 0

        lax.fori_loop(0, CH_E // (2 * LANES), grp_body, 0)
        pltpu.sync_copy(oz, d2z_hbm.at[pl.ds(off, CH_E)])
        pltpu.sync_copy(oq, d2q_hbm.at[pl.ds(off, CH_E)])
        return 0

    lax.fori_loop(0, EPT // CH_E, chunk_body, 0)


_edge_d2 = functools.partial(
    pl.kernel,
    _edge_d2_body,
    out_type=(
        jax.ShapeDtypeStruct((E,), jnp.float32),
        jax.ShapeDtypeStruct((E,), jnp.float32),
    ),
    mesh=_sc_mesh,
    compiler_params=pltpu.CompilerParams(needs_layout_passes=False,
                                        use_tc_tiling_on_sc=False),
    scratch_types=(
        [pltpu.VMEM((N,), jnp.float32)] * 6
        + [pltpu.VMEM((CH_E,), jnp.int32)] * 2
        + [pltpu.VMEM((CH_E,), jnp.float32)] * 2
    ),
)()


def kernel(x, edge_index, batch, pos, a, b, params, epoch):
    src = edge_index[0]
    dst = edge_index[1]
    p = params

    hT = _encode(x, p['enc_W'], p['enc_b'])
    for l in range(L):
        msgT = _segsum(hT, src, dst)
        hT = _gin(hT, msgT, p['gin_W'][l], p['gin_b'][l], p['gin_eps'][l])

    w2p = jnp.zeros((HID, 8), jnp.float32).at[:, :3].set(p['z_W2'])
    b2p = jnp.zeros((8,), jnp.float32).at[:3].set(p['z_b2'])
    graph_feat, ziT8 = _head(hT, batch.reshape(N, 1), p['z_W1'], p['z_b1'],
                             w2p, b2p)

    d2z, d2q = _edge_d2(ziT8[:3], pos.T, src, dst)
    ce, ploss = _loss(d2z, d2q, pos, a, b)

    pred_pos = jnp.zeros((N, 3), x.dtype)
    return pred_pos, graph_feat, ploss.reshape(()), ce.reshape(())
